# baseline (speedup 1.0000x reference)
import jax
import jax.numpy as jnp
from jax import lax
from jax.experimental import pallas as pl
from jax.experimental.pallas import tpu as pltpu
from jax.experimental.pallas import tpu_sc as plsc

N_NODES = 10000
C = 128
N_EDGES = 320000

NC = 2
NS = 16
NW = NC * NS
K = 128
CHUNKS = 2 * (-(-N_EDGES // (NW * K * 2)))
PER_W = CHUNKS * K
E_PAD = PER_W * NW
SLAB = 624
TAIL0 = N_NODES - NS * SLAB


def _sc_body(xT, meta, wgt, out, acc, midx, wbuf, rows,
             gsem0, gsem1, msem0, msem1, wsem0, wsem1):
    cid = lax.axis_index("c")
    sid = lax.axis_index("s")
    wid = cid * NS + sid
    gsems = (gsem0, gsem1)
    msems = (msem0, msem1)
    wsems = (wsem0, wsem1)

    def meta_cp(ch, b):
        return pltpu.make_async_copy(meta.at[wid, ch], midx.at[b], msems[b])

    def wgt_cp(ch, b):
        return pltpu.make_async_copy(wgt.at[wid, ch], wbuf.at[b], wsems[b])

    def gather(ch, b):
        return pltpu.make_async_copy(
            xT.at[midx.at[b, 1]], rows.at[b], gsems[b])

    def zero_row(i, _):
        for j in range(C // 16):
            rows[0, i, pl.ds(16 * j, 16)] = jnp.zeros((16,), jnp.float32)
        return 0
    lax.fori_loop(0, K, zero_row, 0)
    r0 = sid * SLAB
    off = 0
    while off < SLAB:
        n = min(K, SLAB - off)
        pltpu.sync_copy(rows.at[0, pl.ds(0, n)], acc.at[pl.ds(r0 + off, n)])
        off += n

    @pl.when(sid == 0)
    def _zero_tail():
        pltpu.sync_copy(rows.at[0, pl.ds(0, TAIL0)],
                        acc.at[pl.ds(NS * SLAB, TAIL0)])
    plsc.subcore_barrier()

    meta_cp(0, 0).start()
    wgt_cp(0, 0).start()
    meta_cp(1, 1).start()
    wgt_cp(1, 1).start()
    meta_cp(0, 0).wait()
    wgt_cp(0, 0).wait()
    gather(0, 0).start()

    def pair(g, _):
        for b in range(2):
            t = 2 * g + b
            b1 = 1 - b
            gather(t, b).wait()

            @pl.when(t + 1 < CHUNKS)
            def _next_gather():
                meta_cp(t + 1, b1).wait()
                wgt_cp(t + 1, b1).wait()
                gather(t + 1, b1).start()

            def scale(g8, _):
                wv = wbuf[b, pl.ds(16 * g8, 16)]
                w2v = wv * wv
                for l in range(16):
                    e = 16 * g8 + l
                    w2 = w2v[l]
                    for j in range(C // 16):
                        rows[b, e, pl.ds(16 * j, 16)] = (
                            rows[b, e, pl.ds(16 * j, 16)] * w2)
                return 0
            lax.fori_loop(0, K // 16, scale, 0)

            pltpu.sync_copy(rows.at[b], acc.at[midx.at[b, 0]], add=True)

            @pl.when(t + 2 < CHUNKS)
            def _prefetch_meta():
                meta_cp(t + 2, b).start()
                wgt_cp(t + 2, b).start()
        return 0
    lax.fori_loop(0, CHUNKS // 2, pair, 0)

    plsc.subcore_barrier()
    pltpu.sync_copy(acc.at[pl.ds(r0, SLAB)], out.at[cid, pl.ds(r0, SLAB)])

    @pl.when(sid == 0)
    def _write_tail():
        pltpu.sync_copy(acc.at[pl.ds(NS * SLAB, TAIL0)],
                        out.at[cid, pl.ds(NS * SLAB, TAIL0)])


def _combine_body(p_ref, o_ref):
    s = p_ref[0] + p_ref[1]
    o_ref[0] = s.T


_combine = pl.pallas_call(
    _combine_body,
    out_shape=jax.ShapeDtypeStruct((1, C, N_NODES), jnp.float32),
)


def kernel(x, iInd, jInd, W):
    xT = jnp.swapaxes(x[0], 0, 1)
    pad = E_PAD - iInd.shape[0]
    iP = jnp.concatenate([iInd, jnp.zeros((pad,), jnp.int32)])
    jP = jnp.concatenate([jInd, jnp.zeros((pad,), jnp.int32)])
    wP = jnp.concatenate([W, jnp.zeros((pad,), jnp.float32)])
    meta = jnp.concatenate([
        iP.reshape(NW, CHUNKS, 1, K),
        jP.reshape(NW, CHUNKS, 1, K),
    ], axis=2)
    wgt = wP.reshape(NW, CHUNKS, K)

    sc = pl.kernel(
        _sc_body,
        out_type=jax.ShapeDtypeStruct((NC, N_NODES, C), jnp.float32),
        mesh=plsc.VectorSubcoreMesh(core_axis_name="c", subcore_axis_name="s"),
        scratch_types=[
            pltpu.VMEM_SHARED((N_NODES, C), jnp.float32),
            pltpu.VMEM((2, 2, K), jnp.int32),
            pltpu.VMEM((2, K), jnp.float32),
            pltpu.VMEM((2, K, C), jnp.float32),
            pltpu.SemaphoreType.DMA,
            pltpu.SemaphoreType.DMA,
            pltpu.SemaphoreType.DMA,
            pltpu.SemaphoreType.DMA,
            pltpu.SemaphoreType.DMA,
            pltpu.SemaphoreType.DMA,
        ],
    )
    partial = sc(xT, meta, wgt)
    return _combine(partial)


# speedup vs baseline: 2.1418x; 2.1418x over previous
import jax
import jax.numpy as jnp
from jax import lax
from jax.experimental import pallas as pl
from jax.experimental.pallas import tpu as pltpu
from jax.experimental.pallas import tpu_sc as plsc

N_NODES = 10000
C = 128
CH = C // 2
N_EDGES = 320000

NC = 2
NS = 16
K = 128
CHUNKS = 2 * (-(-N_EDGES // (NS * K * 2)))
PER_S = CHUNKS * K
E_PAD = PER_S * NS
SLAB = 624
TAIL0 = N_NODES - NS * SLAB


def _sc_body(xTh, meta, wgt, out, xsp, acc, midx, wbuf, rows,
             gsem0, gsem1, msem0, msem1, wsem0, wsem1):
    cid = lax.axis_index("c")
    sid = lax.axis_index("s")
    gsems = (gsem0, gsem1)
    msems = (msem0, msem1)
    wsems = (wsem0, wsem1)

    def meta_cp(ch, b):
        return pltpu.make_async_copy(meta.at[sid, ch], midx.at[b], msems[b])

    def wgt_cp(ch, b):
        return pltpu.make_async_copy(wgt.at[sid, ch], wbuf.at[b], wsems[b])

    def gather(ch, b):
        return pltpu.make_async_copy(
            xsp.at[midx.at[b, 1]], rows.at[b], gsems[b])

    r0 = sid * SLAB
    pltpu.sync_copy(xTh.at[cid, pl.ds(r0, SLAB)], xsp.at[pl.ds(r0, SLAB)])

    @pl.when(sid == 0)
    def _stage_tail():
        pltpu.sync_copy(xTh.at[cid, pl.ds(NS * SLAB, TAIL0)],
                        xsp.at[pl.ds(NS * SLAB, TAIL0)])

    def zero_row(i, _):
        for j in range(CH // 16):
            rows[0, i, pl.ds(16 * j, 16)] = jnp.zeros((16,), jnp.float32)
        return 0
    lax.fori_loop(0, K, zero_row, 0)
    off = 0
    while off < SLAB:
        n = min(K, SLAB - off)
        pltpu.sync_copy(rows.at[0, pl.ds(0, n)], acc.at[pl.ds(r0 + off, n)])
        off += n

    @pl.when(sid == 0)
    def _zero_tail():
        pltpu.sync_copy(rows.at[0, pl.ds(0, TAIL0)],
                        acc.at[pl.ds(NS * SLAB, TAIL0)])
    plsc.subcore_barrier()

    meta_cp(0, 0).start()
    wgt_cp(0, 0).start()
    meta_cp(1, 1).start()
    wgt_cp(1, 1).start()
    meta_cp(0, 0).wait()
    wgt_cp(0, 0).wait()
    gather(0, 0).start()

    def pair(g, _):
        for b in range(2):
            t = 2 * g + b
            b1 = 1 - b
            gather(t, b).wait()

            @pl.when(t + 1 < CHUNKS)
            def _next_gather():
                meta_cp(t + 1, b1).wait()
                wgt_cp(t + 1, b1).wait()
                gather(t + 1, b1).start()

            def scale(g8, _):
                wv = wbuf[b, pl.ds(16 * g8, 16)]
                w2v = wv * wv
                for l in range(16):
                    e = 16 * g8 + l
                    w2 = w2v[l]
                    for j in range(CH // 16):
                        rows[b, e, pl.ds(16 * j, 16)] = (
                            rows[b, e, pl.ds(16 * j, 16)] * w2)
                return 0
            lax.fori_loop(0, K // 16, scale, 0)

            pltpu.sync_copy(rows.at[b], acc.at[midx.at[b, 0]], add=True)

            @pl.when(t + 2 < CHUNKS)
            def _prefetch_meta():
                meta_cp(t + 2, b).start()
                wgt_cp(t + 2, b).start()
        return 0
    lax.fori_loop(0, CHUNKS // 2, pair, 0)

    plsc.subcore_barrier()
    pltpu.sync_copy(acc.at[pl.ds(r0, SLAB)], out.at[cid, pl.ds(r0, SLAB)])

    @pl.when(sid == 0)
    def _write_tail():
        pltpu.sync_copy(acc.at[pl.ds(NS * SLAB, TAIL0)],
                        out.at[cid, pl.ds(NS * SLAB, TAIL0)])


def _combine_body(p_ref, o_ref):
    o_ref[0] = jnp.concatenate([p_ref[0].T, p_ref[1].T], axis=0)


_combine = pl.pallas_call(
    _combine_body,
    out_shape=jax.ShapeDtypeStruct((1, C, N_NODES), jnp.float32),
)


def kernel(x, iInd, jInd, W):
    xT = jnp.swapaxes(x[0], 0, 1)
    xTh = jnp.stack([xT[:, :CH], xT[:, CH:]])
    pad = E_PAD - iInd.shape[0]
    iP = jnp.concatenate([iInd, jnp.zeros((pad,), jnp.int32)])
    jP = jnp.concatenate([jInd, jnp.zeros((pad,), jnp.int32)])
    wP = jnp.concatenate([W, jnp.zeros((pad,), jnp.float32)])
    meta = jnp.concatenate([
        iP.reshape(NS, CHUNKS, 1, K),
        jP.reshape(NS, CHUNKS, 1, K),
    ], axis=2)
    wgt = wP.reshape(NS, CHUNKS, K)

    sc = pl.kernel(
        _sc_body,
        out_type=jax.ShapeDtypeStruct((NC, N_NODES, CH), jnp.float32),
        mesh=plsc.VectorSubcoreMesh(core_axis_name="c", subcore_axis_name="s"),
        scratch_types=[
            pltpu.VMEM_SHARED((N_NODES, CH), jnp.float32),
            pltpu.VMEM_SHARED((N_NODES, CH), jnp.float32),
            pltpu.VMEM((2, 2, K), jnp.int32),
            pltpu.VMEM((2, K), jnp.float32),
            pltpu.VMEM((2, K, CH), jnp.float32),
            pltpu.SemaphoreType.DMA,
            pltpu.SemaphoreType.DMA,
            pltpu.SemaphoreType.DMA,
            pltpu.SemaphoreType.DMA,
            pltpu.SemaphoreType.DMA,
            pltpu.SemaphoreType.DMA,
        ],
    )
    partial = sc(xTh, meta, wgt)
    return _combine(partial)
